# baseline (speedup 1.0000x reference)
import functools

import jax
import jax.numpy as jnp
from jax import lax
from jax.experimental import pallas as pl
from jax.experimental.pallas import tpu as pltpu
from jax.experimental.pallas import tpu_sc as plsc

_N = 10000
_E = 320000
_D = 128
_NEG = 0.2

_NC = 2
_NS = 16
_NW = _NC * _NS
_EW = _E // _NW
_CE = 80
_NCH = _EW // _CE
_RB = 10
_BN = _N // _RB
_NP = 10240
_BNP = _NP // _RB
_ZR = 40
_ZT = 10

_mesh = plsc.VectorSubcoreMesh(
    core_axis_name="c", subcore_axis_name="s", num_cores=_NC, num_subcores=_NS
)
_sc_params = pltpu.CompilerParams(needs_layout_passes=False)


def _prep_body(h_ref, w_ref, a_ref, r_ref, wh_ref, sp_ref):
    hb = h_ref[...]
    wm = w_ref[...]
    wh = jnp.dot(hb, wm, preferred_element_type=jnp.float32)
    wh_ref[...] = wh
    av = a_ref[...][:, 0]
    a1 = av[0:_D].reshape(1, _D)
    a2 = av[_D:2 * _D].reshape(1, _D)
    i = pl.program_id(0)
    sp_ref[0:1, pl.ds(i * _BNP, _BNP)] = jnp.sum(wh * a1, axis=1).reshape(1, _BNP)
    sp_ref[1:2, pl.ds(i * _BNP, _BNP)] = jnp.sum(wh * a2, axis=1).reshape(1, _BNP)

    @pl.when(i == 0)
    def _sr():
        a3 = av[2 * _D:].reshape(1, _D)
        sr = jnp.sum(r_ref[...] * a3, axis=1).reshape(1, 16)
        sp_ref[2:3, pl.ds(0, 16)] = sr


def _prep(h, W, a, rel_emb):
    return pl.pallas_call(
        _prep_body,
        grid=(_RB,),
        in_specs=[
            pl.BlockSpec((_BNP, _D), lambda i: (i, 0)),
            pl.BlockSpec((_D, _D), lambda i: (0, 0)),
            pl.BlockSpec((3 * _D, 1), lambda i: (0, 0)),
            pl.BlockSpec((16, _D), lambda i: (0, 0)),
        ],
        out_specs=[
            pl.BlockSpec((_BNP, _D), lambda i: (i, 0)),
            pl.BlockSpec((4, _NP), lambda i: (0, 0)),
        ],
        out_shape=[
            jax.ShapeDtypeStruct((_NP, _D), jnp.float32),
            jax.ShapeDtypeStruct((4, _NP), jnp.float32),
        ],
    )(h, W, a, rel_emb)


@functools.partial(
    pl.kernel,
    out_type=[
        jax.ShapeDtypeStruct((2 * _E,), jnp.int32),
        jax.ShapeDtypeStruct((_NW * 16,), jnp.float32),
    ],
    mesh=_mesh,
    compiler_params=_sc_params,
    scratch_types=[
        pltpu.VMEM((_EW,), jnp.int32),
        pltpu.VMEM((_EW,), jnp.int32),
        pltpu.VMEM((_EW,), jnp.int32),
        pltpu.VMEM((2 * _EW,), jnp.int32),
        pltpu.VMEM((_NP,), jnp.float32),
        pltpu.VMEM((_NP,), jnp.float32),
        pltpu.VMEM((16,), jnp.float32),
        pltpu.VMEM((16,), jnp.float32),
        pltpu.SemaphoreType.DMA,
    ],
)
def _escore(rows_hbm, cols_hbm, et_hbm, sp_hbm,
            ce_hbm, mx_hbm,
            rows_v, cols_v, et_v, ce_v, s1_v, s2_v, sr_v, mxs_v, sb):
    cid = lax.axis_index("c")
    sid = lax.axis_index("s")
    wid = sid * _NC + cid
    base = wid * _EW
    pltpu.async_copy(rows_hbm.at[pl.ds(base, _EW)], rows_v, sb)
    pltpu.async_copy(cols_hbm.at[pl.ds(base, _EW)], cols_v, sb)
    pltpu.async_copy(et_hbm.at[pl.ds(base, _EW)], et_v, sb)
    pltpu.async_copy(sp_hbm.at[0], s1_v, sb)
    pltpu.async_copy(sp_hbm.at[1], s2_v, sb)
    pltpu.async_copy(sp_hbm.at[2, pl.ds(0, 16)], sr_v, sb)
    pltpu.make_async_copy(rows_hbm.at[pl.ds(base, _EW)], rows_v, sb).wait()
    pltpu.make_async_copy(cols_hbm.at[pl.ds(base, _EW)], cols_v, sb).wait()
    pltpu.make_async_copy(et_hbm.at[pl.ds(base, _EW)], et_v, sb).wait()
    pltpu.make_async_copy(sp_hbm.at[0], s1_v, sb).wait()
    pltpu.make_async_copy(sp_hbm.at[1], s2_v, sb).wait()
    pltpu.make_async_copy(sp_hbm.at[2, pl.ds(0, 16)], sr_v, sb).wait()

    def _it(i, mx):
        off = pl.multiple_of(i * 16, 16)
        r = rows_v[pl.ds(off, 16)]
        c = cols_v[pl.ds(off, 16)]
        t = et_v[pl.ds(off, 16)]
        x = (plsc.load_gather(s1_v, [r])
             + plsc.load_gather(s2_v, [c])
             + plsc.load_gather(sr_v, [t]))
        e = jnp.where(x > 0.0, x, _NEG * x)
        pk = pl.multiple_of((i // 5) * 160 + (i % 5) * 16, 16)
        ce_v[pl.ds(pk, 16)] = c
        ce_v[pl.ds(pk + 80, 16)] = plsc.bitcast(e, jnp.int32)
        return jnp.maximum(mx, e)

    mx = lax.fori_loop(0, _EW // 16, _it, jnp.full((16,), -3e38, jnp.float32))
    mxs_v[...] = mx
    pltpu.sync_copy(ce_v, ce_hbm.at[pl.ds(2 * base, 2 * _EW)])
    pltpu.sync_copy(mxs_v, mx_hbm.at[pl.ds(wid * 16, 16)])


@functools.partial(
    pl.kernel,
    out_type=[
        jax.ShapeDtypeStruct((_NC, _N, _D), jnp.float32),
        jax.ShapeDtypeStruct((_NC * _N,), jnp.float32),
    ],
    mesh=_mesh,
    compiler_params=_sc_params,
    scratch_types=[
        [pltpu.VMEM((_CE,), jnp.int32)] * 4,
        [pltpu.VMEM((2 * _CE,), jnp.int32)] * 4,
        [pltpu.VMEM((_CE,), jnp.float32)] * 4,
        [pltpu.VMEM((_CE, _D), jnp.float32)] * 4,
        pltpu.VMEM((_BN,), jnp.float32),
        pltpu.VMEM((_NW * 16,), jnp.float32),
        pltpu.VMEM_SHARED((_N, _D), jnp.float32),
        pltpu.VMEM_SHARED((_N,), jnp.float32),
        [pltpu.SemaphoreType.DMA] * 4,
        [pltpu.SemaphoreType.DMA] * 4,
        [pltpu.SemaphoreType.DMA] * 4,
        [pltpu.SemaphoreType.DMA] * 4,
    ],
)
def _accum(rows_hbm, ce_hbm, mx_hbm, wh_hbm, zu_hbm, zs_hbm,
           u_hbm, s_hbm,
           rows_c, cec, exp_c, gbuf, zs_v, mx_v,
           u_sh, s_sh, si, sg, su, ss):
    cid = lax.axis_index("c")
    sid = lax.axis_index("s")
    wid = sid * _NC + cid
    wbase = wid * _EW

    _NSLAB = _N // _CE
    pltpu.sync_copy(zu_hbm, gbuf[0])
    for t in range(8):
        g = sid + 16 * t

        @pl.when(g < _NSLAB)
        def _fz(g=g, t=t):
            pltpu.async_copy(gbuf[0], u_sh.at[pl.ds(g * _CE, _CE)],
                             su[t % 4])
    for t in range(8):
        g = sid + 16 * t

        @pl.when(g < _NSLAB)
        def _wz(g=g, t=t):
            pltpu.make_async_copy(gbuf[0], u_sh.at[pl.ds(g * _CE, _CE)],
                                  su[t % 4]).wait()

    @pl.when(sid < _ZT)
    def _zero_s():
        pltpu.sync_copy(zs_hbm, zs_v)
        pltpu.sync_copy(zs_v, s_sh.at[pl.ds(sid * _BN, _BN)])

    plsc.subcore_barrier()

    pltpu.sync_copy(mx_hbm, mx_v)

    mv = mx_v[pl.ds(0, 16)]
    for i in range(1, _NW):
        mv = jnp.maximum(mv, mx_v[pl.ds(16 * i, 16)])
    m = jnp.max(mv)

    col16 = lax.iota(jnp.int32, 16)

    def _stage(jc, s):
        off = pl.multiple_of(wbase + jc * _CE, 16)
        off2 = pl.multiple_of(2 * wbase + jc * 2 * _CE, 16)
        pltpu.async_copy(rows_hbm.at[pl.ds(off, _CE)], rows_c[s], si[s])
        pltpu.async_copy(ce_hbm.at[pl.ds(off2, 2 * _CE)], cec[s], si[s])

    def _wait_stage(jc, s):
        off = pl.multiple_of(wbase + jc * _CE, 16)
        off2 = pl.multiple_of(2 * wbase + jc * 2 * _CE, 16)
        pltpu.make_async_copy(rows_hbm.at[pl.ds(off, _CE)], rows_c[s],
                              si[s]).wait()
        pltpu.make_async_copy(ce_hbm.at[pl.ds(off2, 2 * _CE)], cec[s],
                              si[s]).wait()

    def _start_gather(s):
        pltpu.async_copy(wh_hbm.at[cec[s].at[pl.ds(0, _CE)]], gbuf[s], sg[s])

    def _wait_gather(s):
        pltpu.make_async_copy(wh_hbm.at[cec[s].at[pl.ds(0, _CE)]], gbuf[s],
                              sg[s]).wait()

    def _start_scatter(s):
        pltpu.async_copy(gbuf[s], u_sh.at[rows_c[s]], su[s], add=True)
        pltpu.async_copy(exp_c[s], s_sh.at[rows_c[s]], ss[s], add=True)

    def _wait_scatter(s):
        pltpu.make_async_copy(gbuf[s], u_sh.at[rows_c[s]], su[s]).wait()
        pltpu.make_async_copy(exp_c[s], s_sh.at[rows_c[s]], ss[s]).wait()

    def _scale(s):
        for k in range(_CE // 16):
            ev = plsc.bitcast(cec[s][pl.ds(_CE + k * 16, 16)], jnp.float32)
            exp_c[s][pl.ds(k * 16, 16)] = jnp.exp(ev - m)

        def _edge(i, c2):
            i0 = 4 * i
            for u in range(4):
                wv = plsc.load_gather(
                    exp_c[s], [jnp.full((16,), i0 + u, jnp.int32)])
                row = gbuf[s].at[i0 + u]
                for k in range(_D // 16):
                    sl = pl.ds(16 * k, 16)
                    row[sl] = row[sl] * wv
            return c2

        lax.fori_loop(0, _CE // 4, _edge, 0)

    def _step(jc, d, last_static):
        s_s = (d + 2) % 4

        if last_static is None:
            @pl.when(jc >= 2)
            def _w():
                _wait_scatter(s_s)

            @pl.when(jc + 2 <= _NCH - 1)
            def _st():
                _stage(jc + 2, s_s)

            @pl.when(jc + 1 <= _NCH - 1)
            def _g():
                _wait_stage(jc + 1, (d + 1) % 4)
                _start_gather((d + 1) % 4)
        else:
            if jc >= 2:
                _wait_scatter(s_s)
            if jc + 2 <= _NCH - 1:
                _stage(jc + 2, s_s)
            if jc + 1 <= _NCH - 1:
                _wait_stage(jc + 1, (d + 1) % 4)
                _start_gather((d + 1) % 4)

        _wait_gather(d)
        _scale(d)
        _start_scatter(d)

    _stage(0, 0)
    _stage(1, 1)
    _wait_stage(0, 0)
    _start_gather(0)

    def _body(i, carry):
        jc = i * 4
        _step(jc, 0, None)
        _step(jc + 1, 1, None)
        _step(jc + 2, 2, None)
        _step(jc + 3, 3, None)
        return carry

    lax.fori_loop(0, (_NCH - 1) // 4, _body, 0)
    _step(_NCH - 1, (_NCH - 1) % 4, True)
    _wait_scatter((_NCH - 2) % 4)
    _wait_scatter((_NCH - 1) % 4)

    plsc.subcore_barrier()

    for t in range(8):
        g = sid + 16 * t
        b = t % 4

        @pl.when(g < _NSLAB)
        def _wo(g=g, b=b, t=t):
            if t >= 4:
                gp = g - 64
                pltpu.make_async_copy(
                    gbuf[b], u_hbm.at[cid, pl.ds(gp * _CE, _CE)], su[b]
                ).wait()
            sl = pl.ds(g * _CE, _CE)
            pltpu.sync_copy(u_sh.at[sl], gbuf[b])
            pltpu.async_copy(gbuf[b], u_hbm.at[cid, sl], su[b])
    for t in range(4, 8):
        g = sid + 16 * t
        b = t % 4

        @pl.when(g < _NSLAB)
        def _wd(g=g, b=b):
            pltpu.make_async_copy(gbuf[b], u_hbm.at[cid, pl.ds(g * _CE, _CE)],
                                  su[b]).wait()

    for t in range(4):
        g = sid + 16 * t
        b = t % 4

        @pl.when((g < _NSLAB) & (g + 64 >= _NSLAB))
        def _wd3(g=g, b=b):
            pltpu.make_async_copy(gbuf[b], u_hbm.at[cid, pl.ds(g * _CE, _CE)],
                                  su[b]).wait()

    @pl.when(sid < _ZT)
    def _out_s():
        sl = pl.ds(sid * _BN, _BN)
        pltpu.sync_copy(s_sh.at[sl], zs_v)
        pltpu.sync_copy(zs_v, s_hbm.at[pl.ds(cid * _N + sid * _BN, _BN)])


def _final_body(u_ref, s_ref, o_ref):
    u = u_ref[0] + u_ref[1]
    s = s_ref[0, 0] + s_ref[0, 1]
    hp = u / (s + 1e-10)[:, None]
    o_ref[...] = jnp.where(hp > 0.0, hp, jnp.exp(jnp.minimum(hp, 0.0)) - 1.0)


def _final(u, s):
    return pl.pallas_call(
        _final_body,
        grid=(_RB,),
        in_specs=[
            pl.BlockSpec((_NC, _BN, _D), lambda i: (0, i, 0)),
            pl.BlockSpec((1, _NC, _BN), lambda i: (i, 0, 0)),
        ],
        out_specs=pl.BlockSpec((_BN, _D), lambda i: (i, 0)),
        out_shape=jax.ShapeDtypeStruct((_N, _D), jnp.float32),
    )(u, s)


@jax.jit
def _impl(h, adj, edge_types, W, rel_emb, a):
    rows = adj[0]
    cols = adj[1]
    h_p = jnp.pad(h, ((0, _NP - _N), (0, 0)))
    wh, sp = _prep(h_p, W, a, rel_emb)
    ce, mx = _escore(rows, cols, edge_types, sp)
    zu = jnp.zeros((_CE, _D), jnp.float32)
    zs = jnp.zeros((_BN,), jnp.float32)
    u, s = _accum(rows, ce, mx, wh, zu, zs)
    s3 = s.reshape(_NC, _RB, _BN).transpose(1, 0, 2)
    return _final(u, s3)


def kernel(h, adj, edge_types, W, rel_emb, a):
    return _impl(h, adj, edge_types, W, rel_emb, a)


# speedup vs baseline: 1.0713x; 1.0713x over previous
import functools

import jax
import jax.numpy as jnp
from jax import lax
from jax.experimental import pallas as pl
from jax.experimental.pallas import tpu as pltpu
from jax.experimental.pallas import tpu_sc as plsc

_N = 10000
_E = 320000
_D = 128
_NEG = 0.2

_NC = 2
_NS = 16
_NW = _NC * _NS
_EW = _E // _NW
_CE = 80
_NCH = _EW // _CE
_RB = 10
_BN = _N // _RB
_NP = 10240
_BNP = _NP // _RB
_ZR = 40
_ZT = 10

_mesh = plsc.VectorSubcoreMesh(
    core_axis_name="c", subcore_axis_name="s", num_cores=_NC, num_subcores=_NS
)
_sc_params = pltpu.CompilerParams(needs_layout_passes=False)


def _prep_body(h_ref, w_ref, a_ref, r_ref, wh_ref, sp_ref):
    hb = h_ref[...]
    wm = w_ref[...]
    wh = jnp.dot(hb, wm, preferred_element_type=jnp.float32)
    wh_ref[...] = wh
    av = a_ref[...][:, 0]
    a1 = av[0:_D].reshape(1, _D)
    a2 = av[_D:2 * _D].reshape(1, _D)
    i = pl.program_id(0)
    sp_ref[0:1, pl.ds(i * _BNP, _BNP)] = jnp.sum(wh * a1, axis=1).reshape(1, _BNP)
    sp_ref[1:2, pl.ds(i * _BNP, _BNP)] = jnp.sum(wh * a2, axis=1).reshape(1, _BNP)

    @pl.when(i == 0)
    def _sr():
        a3 = av[2 * _D:].reshape(1, _D)
        sr = jnp.sum(r_ref[...] * a3, axis=1).reshape(1, 16)
        sp_ref[2:3, pl.ds(0, 16)] = sr


def _prep(h, W, a, rel_emb):
    return pl.pallas_call(
        _prep_body,
        grid=(_RB,),
        in_specs=[
            pl.BlockSpec((_BNP, _D), lambda i: (i, 0)),
            pl.BlockSpec((_D, _D), lambda i: (0, 0)),
            pl.BlockSpec((3 * _D, 1), lambda i: (0, 0)),
            pl.BlockSpec((16, _D), lambda i: (0, 0)),
        ],
        out_specs=[
            pl.BlockSpec((_BNP, _D), lambda i: (i, 0)),
            pl.BlockSpec((4, _NP), lambda i: (0, 0)),
        ],
        out_shape=[
            jax.ShapeDtypeStruct((_N, _D), jnp.float32),
            jax.ShapeDtypeStruct((4, _NP), jnp.float32),
        ],
    )(h, W, a, rel_emb)


@functools.partial(
    pl.kernel,
    out_type=[
        jax.ShapeDtypeStruct((_E,), jnp.float32),
        jax.ShapeDtypeStruct((_NW * 16,), jnp.float32),
    ],
    mesh=_mesh,
    compiler_params=_sc_params,
    scratch_types=[
        pltpu.VMEM((_EW,), jnp.int32),
        pltpu.VMEM((_EW,), jnp.int32),
        pltpu.VMEM((_EW,), jnp.int32),
        pltpu.VMEM((_EW,), jnp.float32),
        pltpu.VMEM((_NP,), jnp.float32),
        pltpu.VMEM((_NP,), jnp.float32),
        pltpu.VMEM((16,), jnp.float32),
        pltpu.VMEM((16,), jnp.float32),
        pltpu.SemaphoreType.DMA,
    ],
)
def _escore(rows_hbm, cols_hbm, et_hbm, sp_hbm,
            e_hbm, mx_hbm,
            rows_v, cols_v, et_v, e_v, s1_v, s2_v, sr_v, mxs_v, sb):
    cid = lax.axis_index("c")
    sid = lax.axis_index("s")
    wid = sid * _NC + cid
    base = wid * _EW
    pltpu.async_copy(rows_hbm.at[pl.ds(base, _EW)], rows_v, sb)
    pltpu.async_copy(cols_hbm.at[pl.ds(base, _EW)], cols_v, sb)
    pltpu.async_copy(et_hbm.at[pl.ds(base, _EW)], et_v, sb)
    pltpu.async_copy(sp_hbm.at[0], s1_v, sb)
    pltpu.async_copy(sp_hbm.at[1], s2_v, sb)
    pltpu.async_copy(sp_hbm.at[2, pl.ds(0, 16)], sr_v, sb)
    pltpu.make_async_copy(rows_hbm.at[pl.ds(base, _EW)], rows_v, sb).wait()
    pltpu.make_async_copy(cols_hbm.at[pl.ds(base, _EW)], cols_v, sb).wait()
    pltpu.make_async_copy(et_hbm.at[pl.ds(base, _EW)], et_v, sb).wait()
    pltpu.make_async_copy(sp_hbm.at[0], s1_v, sb).wait()
    pltpu.make_async_copy(sp_hbm.at[1], s2_v, sb).wait()
    pltpu.make_async_copy(sp_hbm.at[2, pl.ds(0, 16)], sr_v, sb).wait()

    def _it(i, mx):
        off = pl.multiple_of(i * 16, 16)
        r = rows_v[pl.ds(off, 16)]
        c = cols_v[pl.ds(off, 16)]
        t = et_v[pl.ds(off, 16)]
        x = (plsc.load_gather(s1_v, [r])
             + plsc.load_gather(s2_v, [c])
             + plsc.load_gather(sr_v, [t]))
        e = jnp.where(x > 0.0, x, _NEG * x)
        e_v[pl.ds(off, 16)] = e
        return jnp.maximum(mx, e)

    mx = lax.fori_loop(0, _EW // 16, _it, jnp.full((16,), -3e38, jnp.float32))
    mxs_v[...] = mx
    pltpu.sync_copy(e_v, e_hbm.at[pl.ds(base, _EW)])
    pltpu.sync_copy(mxs_v, mx_hbm.at[pl.ds(wid * 16, 16)])


@functools.partial(
    pl.kernel,
    out_type=[
        jax.ShapeDtypeStruct((_NC, _N, _D), jnp.float32),
        jax.ShapeDtypeStruct((_NC * _N,), jnp.float32),
    ],
    mesh=_mesh,
    compiler_params=_sc_params,
    scratch_types=[
        [pltpu.VMEM((_CE,), jnp.int32)] * 4,
        [pltpu.VMEM((_CE,), jnp.int32)] * 4,
        [pltpu.VMEM((_CE,), jnp.float32)] * 4,
        [pltpu.VMEM((_CE,), jnp.float32)] * 4,
        [pltpu.VMEM((_CE, _D), jnp.float32)] * 4,
        pltpu.VMEM((_BN,), jnp.float32),
        pltpu.VMEM((_NW * 16,), jnp.float32),
        pltpu.VMEM_SHARED((_N, _D), jnp.float32),
        pltpu.VMEM_SHARED((_N,), jnp.float32),
        [pltpu.SemaphoreType.DMA] * 4,
        [pltpu.SemaphoreType.DMA] * 4,
        [pltpu.SemaphoreType.DMA] * 4,
        [pltpu.SemaphoreType.DMA] * 4,
    ],
)
def _accum(rows_hbm, cols_hbm, e_hbm, mx_hbm, wh_hbm, zu_hbm, zs_hbm,
           u_hbm, s_hbm,
           rows_c, cols_c, e_c, exp_c, gbuf, zs_v, mx_v,
           u_sh, s_sh, si, sg, su, ss):
    cid = lax.axis_index("c")
    sid = lax.axis_index("s")
    wid = sid * _NC + cid
    wbase = wid * _EW

    _NSLAB = _N // _CE
    pltpu.sync_copy(zu_hbm, gbuf[0])
    for t in range(8):
        g = sid + 16 * t

        @pl.when(g < _NSLAB)
        def _fz(g=g, t=t):
            pltpu.async_copy(gbuf[0], u_sh.at[pl.ds(g * _CE, _CE)],
                             su[t % 4])
    for t in range(8):
        g = sid + 16 * t

        @pl.when(g < _NSLAB)
        def _wz(g=g, t=t):
            pltpu.make_async_copy(gbuf[0], u_sh.at[pl.ds(g * _CE, _CE)],
                                  su[t % 4]).wait()

    @pl.when(sid < _ZT)
    def _zero_s():
        pltpu.sync_copy(zs_hbm, zs_v)
        pltpu.sync_copy(zs_v, s_sh.at[pl.ds(sid * _BN, _BN)])

    plsc.subcore_barrier()

    pltpu.sync_copy(mx_hbm, mx_v)

    mv = mx_v[pl.ds(0, 16)]
    for i in range(1, _NW):
        mv = jnp.maximum(mv, mx_v[pl.ds(16 * i, 16)])
    m = jnp.max(mv)

    col16 = lax.iota(jnp.int32, 16)

    def _stage(jc, s):
        off = pl.multiple_of(wbase + jc * _CE, 16)
        pltpu.async_copy(rows_hbm.at[pl.ds(off, _CE)], rows_c[s], si[s])
        pltpu.async_copy(cols_hbm.at[pl.ds(off, _CE)], cols_c[s], si[s])
        pltpu.async_copy(e_hbm.at[pl.ds(off, _CE)], e_c[s], si[s])

    def _wait_stage(jc, s):
        off = pl.multiple_of(wbase + jc * _CE, 16)
        pltpu.make_async_copy(rows_hbm.at[pl.ds(off, _CE)], rows_c[s],
                              si[s]).wait()
        pltpu.make_async_copy(cols_hbm.at[pl.ds(off, _CE)], cols_c[s],
                              si[s]).wait()
        pltpu.make_async_copy(e_hbm.at[pl.ds(off, _CE)], e_c[s],
                              si[s]).wait()

    def _start_gather(s):
        pltpu.async_copy(wh_hbm.at[cols_c[s]], gbuf[s], sg[s])

    def _wait_gather(s):
        pltpu.make_async_copy(wh_hbm.at[cols_c[s]], gbuf[s], sg[s]).wait()

    def _start_scatter(s):
        pltpu.async_copy(gbuf[s], u_sh.at[rows_c[s]], su[s], add=True)
        pltpu.async_copy(exp_c[s], s_sh.at[rows_c[s]], ss[s], add=True)

    def _wait_scatter(s):
        pltpu.make_async_copy(gbuf[s], u_sh.at[rows_c[s]], su[s]).wait()
        pltpu.make_async_copy(exp_c[s], s_sh.at[rows_c[s]], ss[s]).wait()

    def _scale(s):
        for k in range(_CE // 16):
            exp_c[s][pl.ds(k * 16, 16)] = jnp.exp(e_c[s][pl.ds(k * 16, 16)] - m)

        def _edge(i, c2):
            i0 = 2 * i
            wv0 = plsc.load_gather(exp_c[s], [jnp.full((16,), i0, jnp.int32)])
            wv1 = plsc.load_gather(exp_c[s],
                                   [jnp.full((16,), i0 + 1, jnp.int32)])
            row0 = gbuf[s].at[i0]
            row1 = gbuf[s].at[i0 + 1]
            for k in range(_D // 16):
                sl = pl.ds(16 * k, 16)
                row0[sl] = row0[sl] * wv0
                row1[sl] = row1[sl] * wv1
            return c2

        lax.fori_loop(0, _CE // 2, _edge, 0)

    def _step(jc, d, last_static):
        s_s = (d + 2) % 4

        if last_static is None:
            @pl.when(jc >= 2)
            def _w():
                _wait_scatter(s_s)

            @pl.when(jc + 2 <= _NCH - 1)
            def _st():
                _stage(jc + 2, s_s)

            @pl.when(jc + 1 <= _NCH - 1)
            def _g():
                _wait_stage(jc + 1, (d + 1) % 4)
                _start_gather((d + 1) % 4)
        else:
            if jc >= 2:
                _wait_scatter(s_s)
            if jc + 2 <= _NCH - 1:
                _stage(jc + 2, s_s)
            if jc + 1 <= _NCH - 1:
                _wait_stage(jc + 1, (d + 1) % 4)
                _start_gather((d + 1) % 4)

        _wait_gather(d)
        _scale(d)
        _start_scatter(d)

    _stage(0, 0)
    _stage(1, 1)
    _wait_stage(0, 0)
    _start_gather(0)

    def _body(i, carry):
        jc = i * 4
        _step(jc, 0, None)
        _step(jc + 1, 1, None)
        _step(jc + 2, 2, None)
        _step(jc + 3, 3, None)
        return carry

    lax.fori_loop(0, (_NCH - 1) // 4, _body, 0)
    _step(_NCH - 1, (_NCH - 1) % 4, True)
    _wait_scatter((_NCH - 2) % 4)
    _wait_scatter((_NCH - 1) % 4)

    plsc.subcore_barrier()

    for t in range(8):
        g = sid + 16 * t
        b = t % 4

        @pl.when(g < _NSLAB)
        def _wo(g=g, b=b, t=t):
            if t >= 4:
                gp = g - 64
                pltpu.make_async_copy(
                    gbuf[b], u_hbm.at[cid, pl.ds(gp * _CE, _CE)], su[b]
                ).wait()
            sl = pl.ds(g * _CE, _CE)
            pltpu.sync_copy(u_sh.at[sl], gbuf[b])
            pltpu.async_copy(gbuf[b], u_hbm.at[cid, sl], su[b])
    for t in range(4, 8):
        g = sid + 16 * t
        b = t % 4

        @pl.when(g < _NSLAB)
        def _wd(g=g, b=b):
            pltpu.make_async_copy(gbuf[b], u_hbm.at[cid, pl.ds(g * _CE, _CE)],
                                  su[b]).wait()

    for t in range(4):
        g = sid + 16 * t
        b = t % 4

        @pl.when((g < _NSLAB) & (g + 64 >= _NSLAB))
        def _wd3(g=g, b=b):
            pltpu.make_async_copy(gbuf[b], u_hbm.at[cid, pl.ds(g * _CE, _CE)],
                                  su[b]).wait()

    @pl.when(sid < _ZT)
    def _out_s():
        sl = pl.ds(sid * _BN, _BN)
        pltpu.sync_copy(s_sh.at[sl], zs_v)
        pltpu.sync_copy(zs_v, s_hbm.at[pl.ds(cid * _N + sid * _BN, _BN)])


def _final_body(u_ref, s_ref, o_ref):
    u = u_ref[0] + u_ref[1]
    s = s_ref[0, 0] + s_ref[0, 1]
    hp = u / (s + 1e-10)[:, None]
    o_ref[...] = jnp.where(hp > 0.0, hp, jnp.exp(jnp.minimum(hp, 0.0)) - 1.0)


def _final(u, s):
    return pl.pallas_call(
        _final_body,
        grid=(_RB,),
        in_specs=[
            pl.BlockSpec((_NC, _BN, _D), lambda i: (0, i, 0)),
            pl.BlockSpec((1, _NC, _BN), lambda i: (i, 0, 0)),
        ],
        out_specs=pl.BlockSpec((_BN, _D), lambda i: (i, 0)),
        out_shape=jax.ShapeDtypeStruct((_N, _D), jnp.float32),
    )(u, s)


@jax.jit
def _impl(h, adj, edge_types, W, rel_emb, a):
    rows = adj[0]
    cols = adj[1]
    wh, sp = _prep(h, W, a, rel_emb)
    e, mx = _escore(rows, cols, edge_types, sp)
    zu = jnp.zeros((_CE, _D), jnp.float32)
    zs = jnp.zeros((_BN,), jnp.float32)
    u, s = _accum(rows, cols, e, mx, wh, zu, zs)
    s3 = s.reshape(_NC, _RB, _BN).transpose(1, 0, 2)
    return _final(u, s3)


def kernel(h, adj, edge_types, W, rel_emb, a):
    return _impl(h, adj, edge_types, W, rel_emb, a)


# speedup vs baseline: 1.0819x; 1.0100x over previous
import functools

import jax
import jax.numpy as jnp
from jax import lax
from jax.experimental import pallas as pl
from jax.experimental.pallas import tpu as pltpu
from jax.experimental.pallas import tpu_sc as plsc

_N = 10000
_E = 320000
_D = 128
_NEG = 0.2

_NC = 2
_NS = 16
_NW = _NC * _NS
_EW = _E // _NW
_CE = 80
_NCH = _EW // _CE
_RB = 10
_BN = _N // _RB
_NP = 10240
_BNP = _NP // _RB
_ZR = 40
_ZT = 10

_mesh = plsc.VectorSubcoreMesh(
    core_axis_name="c", subcore_axis_name="s", num_cores=_NC, num_subcores=_NS
)
_sc_params = pltpu.CompilerParams(needs_layout_passes=False)


def _prep_body(h_ref, w_ref, a_ref, r_ref, wh_ref, sp_ref):
    hb = h_ref[...]
    wm = w_ref[...]
    wh = jnp.dot(hb, wm, preferred_element_type=jnp.float32)
    wh_ref[...] = wh
    av = a_ref[...][:, 0]
    a1 = av[0:_D].reshape(1, _D)
    a2 = av[_D:2 * _D].reshape(1, _D)
    i = pl.program_id(0)
    sp_ref[0:1, pl.ds(i * _BNP, _BNP)] = jnp.sum(wh * a1, axis=1).reshape(1, _BNP)
    sp_ref[1:2, pl.ds(i * _BNP, _BNP)] = jnp.sum(wh * a2, axis=1).reshape(1, _BNP)

    @pl.when(i == 0)
    def _sr():
        a3 = av[2 * _D:].reshape(1, _D)
        sr = jnp.sum(r_ref[...] * a3, axis=1).reshape(1, 16)
        sp_ref[2:3, pl.ds(0, 16)] = sr


def _prep(h, W, a, rel_emb):
    return pl.pallas_call(
        _prep_body,
        grid=(_RB,),
        in_specs=[
            pl.BlockSpec((_BNP, _D), lambda i: (i, 0)),
            pl.BlockSpec((_D, _D), lambda i: (0, 0)),
            pl.BlockSpec((3 * _D, 1), lambda i: (0, 0)),
            pl.BlockSpec((16, _D), lambda i: (0, 0)),
        ],
        out_specs=[
            pl.BlockSpec((_BNP, _D), lambda i: (i, 0)),
            pl.BlockSpec((4, _NP), lambda i: (0, 0)),
        ],
        out_shape=[
            jax.ShapeDtypeStruct((_N, _D), jnp.float32),
            jax.ShapeDtypeStruct((4, _NP), jnp.float32),
        ],
    )(h, W, a, rel_emb)


@functools.partial(
    pl.kernel,
    out_type=[
        jax.ShapeDtypeStruct((_E,), jnp.float32),
        jax.ShapeDtypeStruct((_NW * 16,), jnp.float32),
    ],
    mesh=_mesh,
    compiler_params=_sc_params,
    scratch_types=[
        pltpu.VMEM((_EW,), jnp.int32),
        pltpu.VMEM((_EW,), jnp.int32),
        pltpu.VMEM((_EW,), jnp.int32),
        pltpu.VMEM((_EW,), jnp.float32),
        pltpu.VMEM((_NP,), jnp.float32),
        pltpu.VMEM((_NP,), jnp.float32),
        pltpu.VMEM((16,), jnp.float32),
        pltpu.VMEM((16,), jnp.float32),
        pltpu.SemaphoreType.DMA,
    ],
)
def _escore(rows_hbm, cols_hbm, et_hbm, sp_hbm,
            e_hbm, mx_hbm,
            rows_v, cols_v, et_v, e_v, s1_v, s2_v, sr_v, mxs_v, sb):
    cid = lax.axis_index("c")
    sid = lax.axis_index("s")
    wid = sid * _NC + cid
    base = wid * _EW
    pltpu.async_copy(rows_hbm.at[pl.ds(base, _EW)], rows_v, sb)
    pltpu.async_copy(cols_hbm.at[pl.ds(base, _EW)], cols_v, sb)
    pltpu.async_copy(et_hbm.at[pl.ds(base, _EW)], et_v, sb)
    pltpu.async_copy(sp_hbm.at[0], s1_v, sb)
    pltpu.async_copy(sp_hbm.at[1], s2_v, sb)
    pltpu.async_copy(sp_hbm.at[2, pl.ds(0, 16)], sr_v, sb)
    pltpu.make_async_copy(rows_hbm.at[pl.ds(base, _EW)], rows_v, sb).wait()
    pltpu.make_async_copy(cols_hbm.at[pl.ds(base, _EW)], cols_v, sb).wait()
    pltpu.make_async_copy(et_hbm.at[pl.ds(base, _EW)], et_v, sb).wait()
    pltpu.make_async_copy(sp_hbm.at[0], s1_v, sb).wait()
    pltpu.make_async_copy(sp_hbm.at[1], s2_v, sb).wait()
    pltpu.make_async_copy(sp_hbm.at[2, pl.ds(0, 16)], sr_v, sb).wait()

    def _it(i, mx):
        off = pl.multiple_of(i * 16, 16)
        r = rows_v[pl.ds(off, 16)]
        c = cols_v[pl.ds(off, 16)]
        t = et_v[pl.ds(off, 16)]
        x = (plsc.load_gather(s1_v, [r])
             + plsc.load_gather(s2_v, [c])
             + plsc.load_gather(sr_v, [t]))
        e = jnp.where(x > 0.0, x, _NEG * x)
        e_v[pl.ds(off, 16)] = e
        return jnp.maximum(mx, e)

    mx = lax.fori_loop(0, _EW // 16, _it, jnp.full((16,), -3e38, jnp.float32))
    mxs_v[...] = mx
    pltpu.sync_copy(e_v, e_hbm.at[pl.ds(base, _EW)])
    pltpu.sync_copy(mxs_v, mx_hbm.at[pl.ds(wid * 16, 16)])


@functools.partial(
    pl.kernel,
    out_type=[
        jax.ShapeDtypeStruct((_NC, _N, _D), jnp.float32),
        jax.ShapeDtypeStruct((_NC * _N,), jnp.float32),
    ],
    mesh=_mesh,
    compiler_params=_sc_params,
    scratch_types=[
        [pltpu.VMEM((_CE,), jnp.int32)] * 4,
        [pltpu.VMEM((_CE,), jnp.int32)] * 4,
        [pltpu.VMEM((_CE,), jnp.float32)] * 4,
        [pltpu.VMEM((_CE,), jnp.float32)] * 4,
        [pltpu.VMEM((_CE, _D), jnp.float32)] * 4,
        pltpu.VMEM((_BN,), jnp.float32),
        pltpu.VMEM((_NW * 16,), jnp.float32),
        pltpu.VMEM_SHARED((_N, _D), jnp.float32),
        pltpu.VMEM_SHARED((_N,), jnp.float32),
        [pltpu.SemaphoreType.DMA] * 4,
        [pltpu.SemaphoreType.DMA] * 4,
        [pltpu.SemaphoreType.DMA] * 4,
        [pltpu.SemaphoreType.DMA] * 4,
    ],
)
def _accum(rows_hbm, cols_hbm, e_hbm, mx_hbm, wh_hbm, zu_hbm, zs_hbm,
           u_hbm, s_hbm,
           rows_c, cols_c, e_c, exp_c, gbuf, zs_v, mx_v,
           u_sh, s_sh, si, sg, su, ss):
    cid = lax.axis_index("c")
    sid = lax.axis_index("s")
    wid = sid * _NC + cid
    wbase = wid * _EW

    _NSLAB = _N // _CE
    pltpu.sync_copy(zu_hbm, gbuf[0])
    for t in range(8):
        g = sid + 16 * t

        @pl.when(g < _NSLAB)
        def _fz(g=g, t=t):
            pltpu.async_copy(gbuf[0], u_sh.at[pl.ds(g * _CE, _CE)],
                             su[t % 4])
    for t in range(8):
        g = sid + 16 * t

        @pl.when(g < _NSLAB)
        def _wz(g=g, t=t):
            pltpu.make_async_copy(gbuf[0], u_sh.at[pl.ds(g * _CE, _CE)],
                                  su[t % 4]).wait()

    @pl.when(sid < _ZT)
    def _zero_s():
        pltpu.sync_copy(zs_hbm, zs_v)
        pltpu.sync_copy(zs_v, s_sh.at[pl.ds(sid * _BN, _BN)])

    plsc.subcore_barrier()

    pltpu.sync_copy(mx_hbm, mx_v)

    mv = mx_v[pl.ds(0, 16)]
    for i in range(1, _NW):
        mv = jnp.maximum(mv, mx_v[pl.ds(16 * i, 16)])
    m = jnp.max(mv)

    col16 = lax.iota(jnp.int32, 16)

    def _stage(jc, s):
        off = pl.multiple_of(wbase + jc * _CE, 16)
        pltpu.async_copy(rows_hbm.at[pl.ds(off, _CE)], rows_c[s], si[s])
        pltpu.async_copy(cols_hbm.at[pl.ds(off, _CE)], cols_c[s], si[s])
        pltpu.async_copy(e_hbm.at[pl.ds(off, _CE)], e_c[s], si[s])

    def _wait_stage(jc, s):
        off = pl.multiple_of(wbase + jc * _CE, 16)
        pltpu.make_async_copy(rows_hbm.at[pl.ds(off, _CE)], rows_c[s],
                              si[s]).wait()
        pltpu.make_async_copy(cols_hbm.at[pl.ds(off, _CE)], cols_c[s],
                              si[s]).wait()
        pltpu.make_async_copy(e_hbm.at[pl.ds(off, _CE)], e_c[s],
                              si[s]).wait()

    def _start_gather(s):
        pltpu.async_copy(wh_hbm.at[cols_c[s]], gbuf[s], sg[s])

    def _wait_gather(s):
        pltpu.make_async_copy(wh_hbm.at[cols_c[s]], gbuf[s], sg[s]).wait()

    def _start_scatter(s):
        pltpu.async_copy(gbuf[s], u_sh.at[rows_c[s]], su[s], add=True)
        pltpu.async_copy(exp_c[s], s_sh.at[rows_c[s]], ss[s], add=True)

    def _wait_scatter(s):
        pltpu.make_async_copy(gbuf[s], u_sh.at[rows_c[s]], su[s]).wait()
        pltpu.make_async_copy(exp_c[s], s_sh.at[rows_c[s]], ss[s]).wait()

    def _scale(s):
        for k in range(_CE // 16):
            exp_c[s][pl.ds(k * 16, 16)] = jnp.exp(e_c[s][pl.ds(k * 16, 16)] - m)

        def _edge(i, c2):
            i0 = 4 * i
            wvs = [
                plsc.load_gather(exp_c[s], [jnp.full((16,), i0 + u, jnp.int32)])
                for u in range(4)
            ]
            rws = [gbuf[s].at[i0 + u] for u in range(4)]
            for k in range(_D // 16):
                sl = pl.ds(16 * k, 16)
                for u in range(4):
                    rws[u][sl] = rws[u][sl] * wvs[u]
            return c2

        lax.fori_loop(0, _CE // 4, _edge, 0)

    def _step(jc, d, last_static):
        s_s = (d + 2) % 4

        if last_static is None:
            @pl.when(jc >= 2)
            def _w():
                _wait_scatter(s_s)

            @pl.when(jc + 2 <= _NCH - 1)
            def _st():
                _stage(jc + 2, s_s)

            @pl.when(jc + 1 <= _NCH - 1)
            def _g():
                _wait_stage(jc + 1, (d + 1) % 4)
                _start_gather((d + 1) % 4)
        else:
            if jc >= 2:
                _wait_scatter(s_s)
            if jc + 2 <= _NCH - 1:
                _stage(jc + 2, s_s)
            if jc + 1 <= _NCH - 1:
                _wait_stage(jc + 1, (d + 1) % 4)
                _start_gather((d + 1) % 4)

        _wait_gather(d)
        _scale(d)
        _start_scatter(d)

    _stage(0, 0)
    _stage(1, 1)
    _wait_stage(0, 0)
    _start_gather(0)

    def _body(i, carry):
        jc = i * 4
        _step(jc, 0, None)
        _step(jc + 1, 1, None)
        _step(jc + 2, 2, None)
        _step(jc + 3, 3, None)
        return carry

    lax.fori_loop(0, (_NCH - 1) // 4, _body, 0)
    _step(_NCH - 1, (_NCH - 1) % 4, True)
    _wait_scatter((_NCH - 2) % 4)
    _wait_scatter((_NCH - 1) % 4)

    plsc.subcore_barrier()

    for t in range(8):
        g = sid + 16 * t
        b = t % 4

        @pl.when(g < _NSLAB)
        def _wo(g=g, b=b, t=t):
            if t >= 4:
                gp = g - 64
                pltpu.make_async_copy(
                    gbuf[b], u_hbm.at[cid, pl.ds(gp * _CE, _CE)], su[b]
                ).wait()
            sl = pl.ds(g * _CE, _CE)
            pltpu.sync_copy(u_sh.at[sl], gbuf[b])
            pltpu.async_copy(gbuf[b], u_hbm.at[cid, sl], su[b])
    for t in range(4, 8):
        g = sid + 16 * t
        b = t % 4

        @pl.when(g < _NSLAB)
        def _wd(g=g, b=b):
            pltpu.make_async_copy(gbuf[b], u_hbm.at[cid, pl.ds(g * _CE, _CE)],
                                  su[b]).wait()

    for t in range(4):
        g = sid + 16 * t
        b = t % 4

        @pl.when((g < _NSLAB) & (g + 64 >= _NSLAB))
        def _wd3(g=g, b=b):
            pltpu.make_async_copy(gbuf[b], u_hbm.at[cid, pl.ds(g * _CE, _CE)],
                                  su[b]).wait()

    @pl.when(sid < _ZT)
    def _out_s():
        sl = pl.ds(sid * _BN, _BN)
        pltpu.sync_copy(s_sh.at[sl], zs_v)
        pltpu.sync_copy(zs_v, s_hbm.at[pl.ds(cid * _N + sid * _BN, _BN)])


def _final_body(u_ref, s_ref, o_ref):
    u = u_ref[0] + u_ref[1]
    s = s_ref[0, 0] + s_ref[0, 1]
    hp = u / (s + 1e-10)[:, None]
    o_ref[...] = jnp.where(hp > 0.0, hp, jnp.exp(jnp.minimum(hp, 0.0)) - 1.0)


def _final(u, s):
    return pl.pallas_call(
        _final_body,
        grid=(_RB,),
        in_specs=[
            pl.BlockSpec((_NC, _BN, _D), lambda i: (0, i, 0)),
            pl.BlockSpec((1, _NC, _BN), lambda i: (i, 0, 0)),
        ],
        out_specs=pl.BlockSpec((_BN, _D), lambda i: (i, 0)),
        out_shape=jax.ShapeDtypeStruct((_N, _D), jnp.float32),
    )(u, s)


@jax.jit
def _impl(h, adj, edge_types, W, rel_emb, a):
    rows = adj[0]
    cols = adj[1]
    wh, sp = _prep(h, W, a, rel_emb)
    e, mx = _escore(rows, cols, edge_types, sp)
    zu = jnp.zeros((_CE, _D), jnp.float32)
    zs = jnp.zeros((_BN,), jnp.float32)
    u, s = _accum(rows, cols, e, mx, wh, zu, zs)
    s3 = s.reshape(_NC, _RB, _BN).transpose(1, 0, 2)
    return _final(u, s3)


def kernel(h, adj, edge_types, W, rel_emb, a):
    return _impl(h, adj, edge_types, W, rel_emb, a)
